# baseline (speedup 1.0000x reference)
import dataclasses
import functools

import jax
import jax.numpy as jnp
from jax import lax
from jax.experimental import pallas as pl
from jax.experimental.pallas import tpu as pltpu
from jax.experimental.pallas import tpu_sc as plsc

N_NODES = 10000
N_EDGES = 320000
NPAD = 10240
TRASH = 10000
D = 128
BLK = 1280
_GRID = NPAD // BLK

_EB = 128
_NBLK = 80
_SB = 8
_NSB = _NBLK // _SB
_EPT = _NBLK * _EB
_EPAD = 32 * _EPT
_RPT = NPAD // 16
_DROWS = NPAD // 128


def _tc1_body(agg_ref, deg_ref, x_ref, wl_ref, wr_ref, b_ref, o_ref):
    agg = agg_ref[...]
    deg = deg_ref[...]
    mean = agg / jnp.maximum(deg, 1.0)
    x = x_ref[...]
    h = jnp.maximum(mean @ wl_ref[...] + x @ wr_ref[...] + b_ref[...], 0.0)
    o_ref[...] = x + h


def _tc2_body(agg_ref, deg_ref, h1_ref, x_ref, wl_ref, wr_ref, b_ref,
              wres_ref, bres_ref, o_ref):
    agg = agg_ref[...]
    deg = deg_ref[...]
    mean = agg / jnp.maximum(deg, 1.0)
    h1 = h1_ref[...]
    h2 = jnp.maximum(mean @ wl_ref[...] + h1 @ wr_ref[...] + b_ref[...], 0.0)
    h2 = h2 + x_ref[...] @ wres_ref[...] + bres_ref[...]
    dinv = lax.rsqrt(deg_ref[...] + 1.0)
    o_ref[...] = h2 * dinv


def _tc3_body(agg_ref, deg_ref, hn_ref, wcat_ref, bcat_ref, o_ref):
    s = agg_ref[...] + hn_ref[...]
    dinv = lax.rsqrt(deg_ref[...] + 1.0)
    o_ref[...] = (s * dinv) @ wcat_ref[...] + bcat_ref[...]


def _deg_reduce_body(degp_ref, o_ref):
    o_ref[...] = jnp.sum(degp_ref[...], axis=0)


def _row_spec(width):
    return pl.BlockSpec((BLK, width), lambda i: (i, 0))


def _pair_spec(width):
    return pl.BlockSpec((2, BLK, width), lambda i: (0, i, 0))


def _full_spec(shape):
    return pl.BlockSpec(shape, lambda i: tuple(0 for _ in shape))


def _tc1(aggp, deg, x, wl, wr, b):
    return pl.pallas_call(
        _tc1_body,
        grid=(_GRID,),
        in_specs=[_row_spec(D), _row_spec(1), _row_spec(D),
                  _full_spec((D, D)), _full_spec((D, D)), _full_spec((1, D))],
        out_specs=_row_spec(D),
        out_shape=jax.ShapeDtypeStruct((NPAD, D), jnp.float32),
    )(aggp, deg, x, wl, wr, b)


def _tc2(aggp, deg, h1, x, wl, wr, b, wres, bres):
    return pl.pallas_call(
        _tc2_body,
        grid=(_GRID,),
        in_specs=[_row_spec(D), _row_spec(1), _row_spec(D), _row_spec(D),
                  _full_spec((D, D)), _full_spec((D, D)), _full_spec((1, D)),
                  _full_spec((D, D)), _full_spec((1, D))],
        out_specs=_row_spec(D),
        out_shape=jax.ShapeDtypeStruct((NPAD, D), jnp.float32),
    )(aggp, deg, h1, x, wl, wr, b, wres, bres)


def _tc3(aggp, deg, hn, wcat, bcat):
    return pl.pallas_call(
        _tc3_body,
        grid=(_GRID,),
        in_specs=[_row_spec(D), _row_spec(1), _row_spec(D),
                  _full_spec((D, D)), _full_spec((1, D))],
        out_specs=_row_spec(D),
        out_shape=jax.ShapeDtypeStruct((NPAD, D), jnp.float32),
    )(aggp, deg, hn, wcat, bcat)


def _deg_reduce(degp):
    return pl.pallas_call(
        _deg_reduce_body,
        grid=(1,),
        in_specs=[_full_spec((32, _DROWS, 128))],
        out_specs=_full_spec((_DROWS, 128)),
        out_shape=jax.ShapeDtypeStruct((_DROWS, 128), jnp.float32),
    )(degp)


_sc_mesh = plsc.VectorSubcoreMesh(core_axis_name="c", subcore_axis_name="s")

_NBT = 160
_IDXROWS = 16 * _NBT + 16
_WORK_CORE = 0


def _sc_pass_body(vals_hbm, srcb_hbm, dstb_hbm, zeros_hbm, out_hbm,
                  src0, src1, dst0, dst1, row_a, row_b, acc_sh,
                  sem_i0, sem_i1, sem_a, sem_b):
    c = lax.axis_index("c")
    s = lax.axis_index("s")
    base = s * _NBT
    nsb = _NBT // _SB

    srcs = (src0, src1)
    dsts = (dst0, dst1)
    isems = (sem_i0, sem_i1)
    rows = (row_a, row_b)
    rsems = (sem_a, sem_b)

    def idx_copies(sb, p):
        rsl = pl.ds(base + sb * _SB, _SB)
        pltpu.async_copy(srcb_hbm.at[rsl], srcs[p], isems[p])
        pltpu.async_copy(dstb_hbm.at[rsl], dsts[p], isems[p])

    def idx_wait(sb, p):
        rsl = pl.ds(base + sb * _SB, _SB)
        pltpu.make_async_copy(srcb_hbm.at[rsl], srcs[p], isems[p]).wait()
        pltpu.make_async_copy(dstb_hbm.at[rsl], dsts[p], isems[p]).wait()

    def gather(p, b):
        pltpu.async_copy(vals_hbm.at[srcs[p].at[b]], rows[b % 2],
                         rsems[b % 2])

    def gather_wait(p, b):
        pltpu.make_async_copy(vals_hbm.at[srcs[p].at[b]], rows[b % 2],
                              rsems[b % 2]).wait()

    def scatter(p, b):
        pltpu.sync_copy(rows[b % 2], acc_sh.at[dsts[p].at[b]], add=True)

    def process_sb(k, p):
        for b in range(_SB):
            if b < _SB - 1:
                gather(p, b + 1)
            else:
                idx_wait(k + 1, 1 - p)
                gather(1 - p, 0)
            gather_wait(p, b)
            if b == _SB - 1:
                idx_copies(k + 2, p)
            scatter(p, b)

    @pl.when(c == _WORK_CORE)
    def _():
        idx_copies(0, 0)
        zrows = pl.ds(s * _RPT, _RPT)
        pltpu.sync_copy(zeros_hbm.at[zrows], acc_sh.at[zrows])
        plsc.subcore_barrier()

        idx_wait(0, 0)
        idx_copies(1, 1)
        gather(0, 0)

        @pl.loop(0, nsb, step=2)
        def _(j):
            process_sb(j, 0)
            process_sb(j + 1, 1)

        gather_wait(0, 0)
        idx_wait(nsb + 1, 1)

        plsc.subcore_barrier()
        pltpu.sync_copy(acc_sh.at[zrows], out_hbm.at[zrows])


_sc_pass = pl.kernel(
    _sc_pass_body,
    out_type=jax.ShapeDtypeStruct((NPAD, D), jnp.float32),
    mesh=_sc_mesh,
    scratch_types=[
        pltpu.VMEM((_SB, _EB), jnp.int32),
        pltpu.VMEM((_SB, _EB), jnp.int32),
        pltpu.VMEM((_SB, _EB), jnp.int32),
        pltpu.VMEM((_SB, _EB), jnp.int32),
        pltpu.VMEM((_EB, D), jnp.float32),
        pltpu.VMEM((_EB, D), jnp.float32),
        pltpu.VMEM_SHARED((NPAD, D), jnp.float32),
        pltpu.SemaphoreType.DMA,
        pltpu.SemaphoreType.DMA,
        pltpu.SemaphoreType.DMA,
        pltpu.SemaphoreType.DMA,
    ],
)


def _sc_deg_body(dstb_hbm, zeros_hbm, out_hbm, dst_c, deg_tile, sem):
    c = lax.axis_index("c")
    s = lax.axis_index("s")
    wid = c * 16 + s
    base = wid * _NBLK

    pltpu.sync_copy(zeros_hbm.at[pl.ds(0, _DROWS)], deg_tile)
    ones = jnp.ones((16,), jnp.float32)

    @pl.loop(0, _NSB)
    def _(sb):
        pltpu.sync_copy(dstb_hbm.at[pl.ds(base + sb * _SB, _SB)], dst_c)
        for b in range(_SB):
            for k in range(_EB // 16):
                idx = dst_c[b, pl.ds(k * 16, 16)]
                plsc.addupdate_scatter(
                    deg_tile,
                    [lax.shift_right_logical(idx, 7),
                     lax.bitwise_and(idx, 127)],
                    ones)

    pltpu.sync_copy(deg_tile, out_hbm.at[wid])


_sc_cp = pltpu.CompilerParams()
if "needs_layout_passes" in pltpu.CompilerParams.__dataclass_fields__:
    _sc_cp = dataclasses.replace(_sc_cp, needs_layout_passes=False)

_sc_deg = pl.kernel(
    _sc_deg_body,
    out_type=jax.ShapeDtypeStruct((32, _DROWS, 128), jnp.float32),
    mesh=_sc_mesh,
    compiler_params=_sc_cp,
    scratch_types=[
        pltpu.VMEM((_SB, _EB), jnp.int32),
        pltpu.VMEM((_DROWS, 128), jnp.float32),
        pltpu.SemaphoreType.DMA,
    ],
)


def kernel(x, edge_index, W_l1, W_r1, b1, W_l2, W_r2, b2, Wres, bres,
           Wmu, bmu, Wls, bls):
    src = edge_index[0]
    dst = edge_index[1]

    x_pad = jnp.pad(x, ((0, NPAD - N_NODES), (0, 0)))
    b1r = b1.reshape(1, D)
    b2r = b2.reshape(1, D)
    bresr = bres.reshape(1, D)
    wcat = jnp.concatenate([Wmu, Wls], axis=1)
    bcat = jnp.concatenate([bmu, bls]).reshape(1, D)

    pad_e = _EPAD - N_EDGES
    tail = _IDXROWS - 32 * _NBLK
    srcb = jnp.concatenate([src, jnp.zeros((pad_e,), jnp.int32)]).reshape(
        32 * _NBLK, _EB)
    srcb = jnp.concatenate([srcb, jnp.zeros((tail, _EB), jnp.int32)])
    dstb = jnp.concatenate([dst, jnp.full((pad_e,), TRASH, jnp.int32)]).reshape(
        32 * _NBLK, _EB)
    dstb = jnp.concatenate([dstb, jnp.full((tail, _EB), TRASH, jnp.int32)])
    zeros_big = jnp.zeros((NPAD, D), jnp.float32)

    degp = _sc_deg(dstb, zeros_big)
    deg_col = _deg_reduce(degp).reshape(NPAD, 1)

    agg1p = _sc_pass(x_pad, srcb, dstb, zeros_big)
    h1 = _tc1(agg1p, deg_col, x_pad, W_l1, W_r1, b1r)

    agg2p = _sc_pass(h1, srcb, dstb, zeros_big)
    hn = _tc2(agg2p, deg_col, h1, x_pad, W_l2, W_r2, b2r, Wres, bresr)

    agg3p = _sc_pass(hn, srcb, dstb, zeros_big)
    out = _tc3(agg3p, deg_col, hn, wcat, bcat)

    mu = out[:N_NODES, :64]
    logstd = out[:N_NODES, 64:]
    return (mu, logstd)


# speedup vs baseline: 1.0158x; 1.0158x over previous
import dataclasses
import functools

import jax
import jax.numpy as jnp
from jax import lax
from jax.experimental import pallas as pl
from jax.experimental.pallas import tpu as pltpu
from jax.experimental.pallas import tpu_sc as plsc

N_NODES = 10000
N_EDGES = 320000
NPAD = 10240
TRASH = 10000
D = 128
BLK = 1280
_GRID = NPAD // BLK

_EB = 128
_NBLK = 80
_SB = 8
_NSB = _NBLK // _SB
_EPT = _NBLK * _EB
_EPAD = 32 * _EPT
_RPT = NPAD // 16
_DROWS = NPAD // 128


def _tc1_body(agg_ref, deg_ref, x_ref, wl_ref, wr_ref, b_ref, o_ref):
    agg = agg_ref[...]
    deg = deg_ref[...]
    mean = agg / jnp.maximum(deg, 1.0)
    x = x_ref[...]
    h = jnp.maximum(mean @ wl_ref[...] + x @ wr_ref[...] + b_ref[...], 0.0)
    o_ref[...] = x + h


def _tc2_body(agg_ref, deg_ref, h1_ref, x_ref, wl_ref, wr_ref, b_ref,
              wres_ref, bres_ref, o_ref):
    agg = agg_ref[...]
    deg = deg_ref[...]
    mean = agg / jnp.maximum(deg, 1.0)
    h1 = h1_ref[...]
    h2 = jnp.maximum(mean @ wl_ref[...] + h1 @ wr_ref[...] + b_ref[...], 0.0)
    h2 = h2 + x_ref[...] @ wres_ref[...] + bres_ref[...]
    dinv = lax.rsqrt(deg_ref[...] + 1.0)
    o_ref[...] = h2 * dinv


def _tc3_body(agg_ref, deg_ref, hn_ref, wcat_ref, bcat_ref, o_ref):
    s = agg_ref[...] + hn_ref[...]
    dinv = lax.rsqrt(deg_ref[...] + 1.0)
    o_ref[...] = (s * dinv) @ wcat_ref[...] + bcat_ref[...]


def _deg_reduce_body(degp_ref, o_ref):
    o_ref[...] = jnp.sum(degp_ref[...], axis=0)


def _row_spec(width):
    return pl.BlockSpec((BLK, width), lambda i: (i, 0))


def _pair_spec(width):
    return pl.BlockSpec((2, BLK, width), lambda i: (0, i, 0))


def _full_spec(shape):
    return pl.BlockSpec(shape, lambda i: tuple(0 for _ in shape))


def _tc1(aggp, deg, x, wl, wr, b):
    return pl.pallas_call(
        _tc1_body,
        grid=(_GRID,),
        in_specs=[_row_spec(D), _row_spec(1), _row_spec(D),
                  _full_spec((D, D)), _full_spec((D, D)), _full_spec((1, D))],
        out_specs=_row_spec(D),
        out_shape=jax.ShapeDtypeStruct((NPAD, D), jnp.float32),
    )(aggp, deg, x, wl, wr, b)


def _tc2(aggp, deg, h1, x, wl, wr, b, wres, bres):
    return pl.pallas_call(
        _tc2_body,
        grid=(_GRID,),
        in_specs=[_row_spec(D), _row_spec(1), _row_spec(D), _row_spec(D),
                  _full_spec((D, D)), _full_spec((D, D)), _full_spec((1, D)),
                  _full_spec((D, D)), _full_spec((1, D))],
        out_specs=_row_spec(D),
        out_shape=jax.ShapeDtypeStruct((NPAD, D), jnp.float32),
    )(aggp, deg, h1, x, wl, wr, b, wres, bres)


def _tc3(aggp, deg, hn, wcat, bcat):
    return pl.pallas_call(
        _tc3_body,
        grid=(_GRID,),
        in_specs=[_row_spec(D), _row_spec(1), _row_spec(D),
                  _full_spec((D, D)), _full_spec((1, D))],
        out_specs=_row_spec(D),
        out_shape=jax.ShapeDtypeStruct((NPAD, D), jnp.float32),
    )(aggp, deg, hn, wcat, bcat)


def _deg_reduce(degp):
    return pl.pallas_call(
        _deg_reduce_body,
        grid=(1,),
        in_specs=[_full_spec((32, _DROWS, 128))],
        out_specs=_full_spec((_DROWS, 128)),
        out_shape=jax.ShapeDtypeStruct((_DROWS, 128), jnp.float32),
    )(degp)


_sc_mesh = plsc.VectorSubcoreMesh(core_axis_name="c", subcore_axis_name="s")

_NBT = 160
_IDXROWS = 16 * _NBT + 16
_WORK_CORE = 1


def _sc_pass_body(vals_hbm, srcb_hbm, dstb_hbm, zeros_hbm, out_hbm,
                  src0, src1, dst0, dst1, row_a, row_b, acc_sh,
                  sem_i0, sem_i1, sem_a, sem_b):
    c = lax.axis_index("c")
    s = lax.axis_index("s")
    base = s * _NBT
    nsb = _NBT // _SB

    srcs = (src0, src1)
    dsts = (dst0, dst1)
    isems = (sem_i0, sem_i1)
    rows = (row_a, row_b)
    rsems = (sem_a, sem_b)

    def idx_copies(sb, p):
        rsl = pl.ds(base + sb * _SB, _SB)
        pltpu.async_copy(srcb_hbm.at[rsl], srcs[p], isems[p])
        pltpu.async_copy(dstb_hbm.at[rsl], dsts[p], isems[p])

    def idx_wait(sb, p):
        rsl = pl.ds(base + sb * _SB, _SB)
        pltpu.make_async_copy(srcb_hbm.at[rsl], srcs[p], isems[p]).wait()
        pltpu.make_async_copy(dstb_hbm.at[rsl], dsts[p], isems[p]).wait()

    def gather(p, b):
        pltpu.async_copy(vals_hbm.at[srcs[p].at[b]], rows[b % 2],
                         rsems[b % 2])

    def gather_wait(p, b):
        pltpu.make_async_copy(vals_hbm.at[srcs[p].at[b]], rows[b % 2],
                              rsems[b % 2]).wait()

    def scatter(p, b):
        pltpu.sync_copy(rows[b % 2], acc_sh.at[dsts[p].at[b]], add=True)

    def process_sb(k, p):
        for b in range(_SB):
            if b < _SB - 1:
                gather(p, b + 1)
            else:
                idx_wait(k + 1, 1 - p)
                gather(1 - p, 0)
            gather_wait(p, b)
            if b == _SB - 1:
                idx_copies(k + 2, p)
            scatter(p, b)

    @pl.when(c == _WORK_CORE)
    def _():
        idx_copies(0, 0)
        zrows = pl.ds(s * _RPT, _RPT)
        pltpu.sync_copy(zeros_hbm.at[zrows], acc_sh.at[zrows])
        plsc.subcore_barrier()

        idx_wait(0, 0)
        idx_copies(1, 1)
        gather(0, 0)

        @pl.loop(0, nsb, step=2)
        def _(j):
            process_sb(j, 0)
            process_sb(j + 1, 1)

        gather_wait(0, 0)
        idx_wait(nsb + 1, 1)

        plsc.subcore_barrier()
        pltpu.sync_copy(acc_sh.at[zrows], out_hbm.at[zrows])


_sc_pass = pl.kernel(
    _sc_pass_body,
    out_type=jax.ShapeDtypeStruct((NPAD, D), jnp.float32),
    mesh=_sc_mesh,
    scratch_types=[
        pltpu.VMEM((_SB, _EB), jnp.int32),
        pltpu.VMEM((_SB, _EB), jnp.int32),
        pltpu.VMEM((_SB, _EB), jnp.int32),
        pltpu.VMEM((_SB, _EB), jnp.int32),
        pltpu.VMEM((_EB, D), jnp.float32),
        pltpu.VMEM((_EB, D), jnp.float32),
        pltpu.VMEM_SHARED((NPAD, D), jnp.float32),
        pltpu.SemaphoreType.DMA,
        pltpu.SemaphoreType.DMA,
        pltpu.SemaphoreType.DMA,
        pltpu.SemaphoreType.DMA,
    ],
)


def _sc_deg_body(dstb_hbm, zeros_hbm, out_hbm, dst_c, deg_tile, sem):
    c = lax.axis_index("c")
    s = lax.axis_index("s")
    wid = c * 16 + s
    base = wid * _NBLK

    pltpu.sync_copy(zeros_hbm.at[pl.ds(0, _DROWS)], deg_tile)
    ones = jnp.ones((16,), jnp.float32)

    @pl.loop(0, _NSB)
    def _(sb):
        pltpu.sync_copy(dstb_hbm.at[pl.ds(base + sb * _SB, _SB)], dst_c)
        for b in range(_SB):
            for k in range(_EB // 16):
                idx = dst_c[b, pl.ds(k * 16, 16)]
                plsc.addupdate_scatter(
                    deg_tile,
                    [lax.shift_right_logical(idx, 7),
                     lax.bitwise_and(idx, 127)],
                    ones)

    pltpu.sync_copy(deg_tile, out_hbm.at[wid])


_sc_cp = pltpu.CompilerParams()
if "needs_layout_passes" in pltpu.CompilerParams.__dataclass_fields__:
    _sc_cp = dataclasses.replace(_sc_cp, needs_layout_passes=False)

_sc_deg = pl.kernel(
    _sc_deg_body,
    out_type=jax.ShapeDtypeStruct((32, _DROWS, 128), jnp.float32),
    mesh=_sc_mesh,
    compiler_params=_sc_cp,
    scratch_types=[
        pltpu.VMEM((_SB, _EB), jnp.int32),
        pltpu.VMEM((_DROWS, 128), jnp.float32),
        pltpu.SemaphoreType.DMA,
    ],
)


def kernel(x, edge_index, W_l1, W_r1, b1, W_l2, W_r2, b2, Wres, bres,
           Wmu, bmu, Wls, bls):
    src = edge_index[0]
    dst = edge_index[1]

    x_pad = jnp.pad(x, ((0, NPAD - N_NODES), (0, 0)))
    b1r = b1.reshape(1, D)
    b2r = b2.reshape(1, D)
    bresr = bres.reshape(1, D)
    wcat = jnp.concatenate([Wmu, Wls], axis=1)
    bcat = jnp.concatenate([bmu, bls]).reshape(1, D)

    pad_e = _EPAD - N_EDGES
    tail = _IDXROWS - 32 * _NBLK
    srcb = jnp.concatenate([src, jnp.zeros((pad_e,), jnp.int32)]).reshape(
        32 * _NBLK, _EB)
    srcb = jnp.concatenate([srcb, jnp.zeros((tail, _EB), jnp.int32)])
    dstb = jnp.concatenate([dst, jnp.full((pad_e,), TRASH, jnp.int32)]).reshape(
        32 * _NBLK, _EB)
    dstb = jnp.concatenate([dstb, jnp.full((tail, _EB), TRASH, jnp.int32)])
    zeros_big = jnp.zeros((NPAD, D), jnp.float32)

    degp = _sc_deg(dstb, zeros_big)
    deg_col = _deg_reduce(degp).reshape(NPAD, 1)

    agg1p = _sc_pass(x_pad, srcb, dstb, zeros_big)
    h1 = _tc1(agg1p, deg_col, x_pad, W_l1, W_r1, b1r)

    agg2p = _sc_pass(h1, srcb, dstb, zeros_big)
    hn = _tc2(agg2p, deg_col, h1, x_pad, W_l2, W_r2, b2r, Wres, bresr)

    agg3p = _sc_pass(hn, srcb, dstb, zeros_big)
    out = _tc3(agg3p, deg_col, hn, wcat, bcat)

    mu = out[:N_NODES, :64]
    logstd = out[:N_NODES, 64:]
    return (mu, logstd)


# speedup vs baseline: 2.3415x; 2.3051x over previous
import dataclasses
import functools

import jax
import jax.numpy as jnp
from jax import lax
from jax.experimental import pallas as pl
from jax.experimental.pallas import tpu as pltpu
from jax.experimental.pallas import tpu_sc as plsc

N_NODES = 10000
N_EDGES = 320000
NPAD = 10240
TRASH = 10000
D = 128
BLK = 1280
_GRID = NPAD // BLK

_EB = 128
_NBLK = 80
_SB = 8
_NSB = _NBLK // _SB
_EPT = _NBLK * _EB
_EPAD = 32 * _EPT
_RPT = NPAD // 16
_DROWS = NPAD // 128


def _tc1_body(agg_ref, deg_ref, x_ref, wl_ref, wr_ref, b_ref, o_ref):
    agg = agg_ref[...]
    deg = deg_ref[...]
    mean = agg / jnp.maximum(deg, 1.0)
    x = x_ref[...]
    h = jnp.maximum(mean @ wl_ref[...] + x @ wr_ref[...] + b_ref[...], 0.0)
    o_ref[...] = x + h


def _tc2_body(agg_ref, deg_ref, h1_ref, x_ref, wl_ref, wr_ref, b_ref,
              wres_ref, bres_ref, o_ref):
    agg = agg_ref[...]
    deg = deg_ref[...]
    mean = agg / jnp.maximum(deg, 1.0)
    h1 = h1_ref[...]
    h2 = jnp.maximum(mean @ wl_ref[...] + h1 @ wr_ref[...] + b_ref[...], 0.0)
    h2 = h2 + x_ref[...] @ wres_ref[...] + bres_ref[...]
    dinv = lax.rsqrt(deg_ref[...] + 1.0)
    o_ref[...] = h2 * dinv


def _tc3_body(agg_ref, deg_ref, hn_ref, wcat_ref, bcat_ref, o_ref):
    s = agg_ref[...] + hn_ref[...]
    dinv = lax.rsqrt(deg_ref[...] + 1.0)
    o_ref[...] = (s * dinv) @ wcat_ref[...] + bcat_ref[...]


def _deg_reduce_body(degp_ref, o_ref):
    o_ref[...] = jnp.sum(degp_ref[...], axis=0)


def _row_spec(width):
    return pl.BlockSpec((BLK, width), lambda i: (i, 0))


def _pair_spec(width):
    return pl.BlockSpec((2, BLK, width), lambda i: (0, i, 0))


def _full_spec(shape):
    return pl.BlockSpec(shape, lambda i: tuple(0 for _ in shape))


def _tc1(aggp, deg, x, wl, wr, b):
    return pl.pallas_call(
        _tc1_body,
        grid=(_GRID,),
        in_specs=[_row_spec(D), _row_spec(1), _row_spec(D),
                  _full_spec((D, D)), _full_spec((D, D)), _full_spec((1, D))],
        out_specs=_row_spec(D),
        out_shape=jax.ShapeDtypeStruct((NPAD, D), jnp.float32),
    )(aggp, deg, x, wl, wr, b)


def _tc2(aggp, deg, h1, x, wl, wr, b, wres, bres):
    return pl.pallas_call(
        _tc2_body,
        grid=(_GRID,),
        in_specs=[_row_spec(D), _row_spec(1), _row_spec(D), _row_spec(D),
                  _full_spec((D, D)), _full_spec((D, D)), _full_spec((1, D)),
                  _full_spec((D, D)), _full_spec((1, D))],
        out_specs=_row_spec(D),
        out_shape=jax.ShapeDtypeStruct((NPAD, D), jnp.float32),
    )(aggp, deg, h1, x, wl, wr, b, wres, bres)


def _tc3(aggp, deg, hn, wcat, bcat):
    return pl.pallas_call(
        _tc3_body,
        grid=(_GRID,),
        in_specs=[_row_spec(D), _row_spec(1), _row_spec(D),
                  _full_spec((D, D)), _full_spec((1, D))],
        out_specs=_row_spec(D),
        out_shape=jax.ShapeDtypeStruct((NPAD, D), jnp.float32),
    )(aggp, deg, hn, wcat, bcat)


def _deg_reduce(degp):
    return pl.pallas_call(
        _deg_reduce_body,
        grid=(1,),
        in_specs=[_full_spec((32, _DROWS, 128))],
        out_specs=_full_spec((_DROWS, 128)),
        out_shape=jax.ShapeDtypeStruct((_DROWS, 128), jnp.float32),
    )(degp)


_sc_mesh = plsc.VectorSubcoreMesh(core_axis_name="c", subcore_axis_name="s")

_NBT = 160
_IDXROWS = 16 * _NBT + 16
_WORK_CORE = 1


def _sc_pass_body(vals_hbm, srcb_hbm, dstb_hbm, zeros_hbm, out_hbm,
                  src0, src1, dst0, dst1, row_a, row_b, acc_sh,
                  sem_i0, sem_i1, sem_a, sem_b):
    c = lax.axis_index("c")
    s = lax.axis_index("s")
    base = s * _NBT
    nsb = _NBT // _SB

    srcs = (src0, src1)
    dsts = (dst0, dst1)
    isems = (sem_i0, sem_i1)
    rows = (row_a, row_b)
    rsems = (sem_a, sem_b)

    def idx_copies(sb, p):
        rsl = pl.ds(base + sb * _SB, _SB)
        pltpu.async_copy(srcb_hbm.at[rsl], srcs[p], isems[p])
        pltpu.async_copy(dstb_hbm.at[rsl], dsts[p], isems[p])

    def idx_wait(sb, p):
        rsl = pl.ds(base + sb * _SB, _SB)
        pltpu.make_async_copy(srcb_hbm.at[rsl], srcs[p], isems[p]).wait()
        pltpu.make_async_copy(dstb_hbm.at[rsl], dsts[p], isems[p]).wait()

    def gather(p, b):
        pltpu.async_copy(vals_hbm.at[srcs[p].at[b]], rows[b % 2],
                         rsems[b % 2])

    def gather_wait(p, b):
        pltpu.make_async_copy(vals_hbm.at[srcs[p].at[b]], rows[b % 2],
                              rsems[b % 2]).wait()

    def scatter(p, b):
        pltpu.sync_copy(rows[b % 2], acc_sh.at[dsts[p].at[b]], add=True)

    def process_sb(k, p):
        for b in range(_SB):
            if b < _SB - 1:
                gather(p, b + 1)
            else:
                idx_wait(k + 1, 1 - p)
                gather(1 - p, 0)
            gather_wait(p, b)
            if b == _SB - 1:
                idx_copies(k + 2, p)
            scatter(p, b)

    @pl.when(c == _WORK_CORE)
    def _():
        idx_copies(0, 0)
        zrows = pl.ds(s * _RPT, _RPT)
        pltpu.sync_copy(zeros_hbm.at[zrows], acc_sh.at[zrows])
        plsc.subcore_barrier()

        idx_wait(0, 0)
        idx_copies(1, 1)
        gather(0, 0)

        @pl.loop(0, nsb, step=2)
        def _(j):
            process_sb(j, 0)
            process_sb(j + 1, 1)

        gather_wait(0, 0)
        idx_wait(nsb + 1, 1)

        plsc.subcore_barrier()
        pltpu.sync_copy(acc_sh.at[zrows], out_hbm.at[zrows])


_sc_pass = pl.kernel(
    _sc_pass_body,
    out_type=jax.ShapeDtypeStruct((NPAD, D), jnp.float32),
    mesh=_sc_mesh,
    scratch_types=[
        pltpu.VMEM((_SB, _EB), jnp.int32),
        pltpu.VMEM((_SB, _EB), jnp.int32),
        pltpu.VMEM((_SB, _EB), jnp.int32),
        pltpu.VMEM((_SB, _EB), jnp.int32),
        pltpu.VMEM((_EB, D), jnp.float32),
        pltpu.VMEM((_EB, D), jnp.float32),
        pltpu.VMEM_SHARED((NPAD, D), jnp.float32),
        pltpu.SemaphoreType.DMA,
        pltpu.SemaphoreType.DMA,
        pltpu.SemaphoreType.DMA,
        pltpu.SemaphoreType.DMA,
    ],
)


def _sc_deg_body(dstb_hbm, zeros_hbm, out_hbm, dst_c, deg_tile, sem):
    c = lax.axis_index("c")
    s = lax.axis_index("s")
    wid = c * 16 + s
    base = wid * _NBLK

    pltpu.sync_copy(zeros_hbm.at[pl.ds(0, _DROWS)], deg_tile)
    ones = jnp.ones((16,), jnp.float32)

    @pl.loop(0, _NSB)
    def _(sb):
        pltpu.sync_copy(dstb_hbm.at[pl.ds(base + sb * _SB, _SB)], dst_c)
        for b in range(_SB):
            for k in range(_EB // 16):
                idx = dst_c[b, pl.ds(k * 16, 16)]
                plsc.addupdate_scatter(
                    deg_tile,
                    [lax.shift_right_logical(idx, 7),
                     lax.bitwise_and(idx, 127)],
                    ones)

    pltpu.sync_copy(deg_tile, out_hbm.at[wid])


_sc_cp = pltpu.CompilerParams()
if "needs_layout_passes" in pltpu.CompilerParams.__dataclass_fields__:
    _sc_cp = dataclasses.replace(_sc_cp, needs_layout_passes=False)

_sc_deg = pl.kernel(
    _sc_deg_body,
    out_type=jax.ShapeDtypeStruct((32, _DROWS, 128), jnp.float32),
    mesh=_sc_mesh,
    compiler_params=_sc_cp,
    scratch_types=[
        pltpu.VMEM((_SB, _EB), jnp.int32),
        pltpu.VMEM((_DROWS, 128), jnp.float32),
        pltpu.SemaphoreType.DMA,
    ],
)


def kernel(x, edge_index, W_l1, W_r1, b1, W_l2, W_r2, b2, Wres, bres,
           Wmu, bmu, Wls, bls):
    src = edge_index[0]
    dst = edge_index[1]

    x_pad = jnp.pad(x, ((0, NPAD - N_NODES), (0, 0)))
    b1r = b1.reshape(1, D)
    b2r = b2.reshape(1, D)
    bresr = bres.reshape(1, D)
    wcat = jnp.concatenate([Wmu, Wls], axis=1)
    bcat = jnp.concatenate([bmu, bls]).reshape(1, D)

    pad_e = _EPAD - N_EDGES
    tail = _IDXROWS - 32 * _NBLK
    pad_ids = jnp.arange(pad_e, dtype=jnp.int32)
    pad_src = pad_ids % N_NODES
    pad_dst = pad_ids % (NPAD - N_NODES) + TRASH
    srcb = jnp.concatenate([src, pad_src]).reshape(32 * _NBLK, _EB)
    srcb = jnp.concatenate([srcb, jnp.zeros((tail, _EB), jnp.int32)])
    dstb = jnp.concatenate([dst, pad_dst]).reshape(32 * _NBLK, _EB)
    dstb = jnp.concatenate([dstb, jnp.full((tail, _EB), TRASH, jnp.int32)])
    zeros_big = jnp.zeros((NPAD, D), jnp.float32)

    degp = _sc_deg(dstb, zeros_big)
    deg_col = _deg_reduce(degp).reshape(NPAD, 1)

    agg1p = _sc_pass(x_pad, srcb, dstb, zeros_big)
    h1 = _tc1(agg1p, deg_col, x_pad, W_l1, W_r1, b1r)

    agg2p = _sc_pass(h1, srcb, dstb, zeros_big)
    hn = _tc2(agg2p, deg_col, h1, x_pad, W_l2, W_r2, b2r, Wres, bresr)

    agg3p = _sc_pass(hn, srcb, dstb, zeros_big)
    out = _tc3(agg3p, deg_col, hn, wcat, bcat)

    mu = out[:N_NODES, :64]
    logstd = out[:N_NODES, 64:]
    return (mu, logstd)


# speedup vs baseline: 3.7086x; 1.5839x over previous
import dataclasses
import functools

import jax
import jax.numpy as jnp
from jax import lax
from jax.experimental import pallas as pl
from jax.experimental.pallas import tpu as pltpu
from jax.experimental.pallas import tpu_sc as plsc

N_NODES = 10000
N_EDGES = 320000
NPAD = 10240
TRASH = 10000
D = 128
BLK = 1280
_GRID = NPAD // BLK

_EB = 128
_NBLK = 80
_SB = 8
_NSB = _NBLK // _SB
_EPT = _NBLK * _EB
_EPAD = 32 * _EPT
_RPT = NPAD // 16
_DROWS = NPAD // 128


def _tc1_body(agg_ref, deg_ref, x_ref, wl_ref, wr_ref, b_ref, o_ref):
    agg = agg_ref[0] + agg_ref[1]
    deg = deg_ref[...]
    mean = agg / jnp.maximum(deg, 1.0)
    x = x_ref[...]
    h = jnp.maximum(mean @ wl_ref[...] + x @ wr_ref[...] + b_ref[...], 0.0)
    o_ref[...] = x + h


def _tc2_body(agg_ref, deg_ref, h1_ref, x_ref, wl_ref, wr_ref, b_ref,
              wres_ref, bres_ref, o_ref):
    agg = agg_ref[0] + agg_ref[1]
    deg = deg_ref[...]
    mean = agg / jnp.maximum(deg, 1.0)
    h1 = h1_ref[...]
    h2 = jnp.maximum(mean @ wl_ref[...] + h1 @ wr_ref[...] + b_ref[...], 0.0)
    h2 = h2 + x_ref[...] @ wres_ref[...] + bres_ref[...]
    dinv = lax.rsqrt(deg_ref[...] + 1.0)
    o_ref[...] = h2 * dinv


def _tc3_body(agg_ref, deg_ref, hn_ref, wcat_ref, bcat_ref, o_ref):
    s = agg_ref[0] + agg_ref[1] + hn_ref[...]
    dinv = lax.rsqrt(deg_ref[...] + 1.0)
    o_ref[...] = (s * dinv) @ wcat_ref[...] + bcat_ref[...]


def _deg_reduce_body(degp_ref, o_ref):
    o_ref[...] = jnp.sum(degp_ref[...], axis=0)


def _row_spec(width):
    return pl.BlockSpec((BLK, width), lambda i: (i, 0))


def _pair_spec(width):
    return pl.BlockSpec((2, BLK, width), lambda i: (0, i, 0))


def _full_spec(shape):
    return pl.BlockSpec(shape, lambda i: tuple(0 for _ in shape))


def _tc1(aggp, deg, x, wl, wr, b):
    return pl.pallas_call(
        _tc1_body,
        grid=(_GRID,),
        in_specs=[_pair_spec(D), _row_spec(1), _row_spec(D),
                  _full_spec((D, D)), _full_spec((D, D)), _full_spec((1, D))],
        out_specs=_row_spec(D),
        out_shape=jax.ShapeDtypeStruct((NPAD, D), jnp.float32),
    )(aggp, deg, x, wl, wr, b)


def _tc2(aggp, deg, h1, x, wl, wr, b, wres, bres):
    return pl.pallas_call(
        _tc2_body,
        grid=(_GRID,),
        in_specs=[_pair_spec(D), _row_spec(1), _row_spec(D), _row_spec(D),
                  _full_spec((D, D)), _full_spec((D, D)), _full_spec((1, D)),
                  _full_spec((D, D)), _full_spec((1, D))],
        out_specs=_row_spec(D),
        out_shape=jax.ShapeDtypeStruct((NPAD, D), jnp.float32),
    )(aggp, deg, h1, x, wl, wr, b, wres, bres)


def _tc3(aggp, deg, hn, wcat, bcat):
    return pl.pallas_call(
        _tc3_body,
        grid=(_GRID,),
        in_specs=[_pair_spec(D), _row_spec(1), _row_spec(D),
                  _full_spec((D, D)), _full_spec((1, D))],
        out_specs=_row_spec(D),
        out_shape=jax.ShapeDtypeStruct((NPAD, D), jnp.float32),
    )(aggp, deg, hn, wcat, bcat)


def _deg_reduce(degp):
    return pl.pallas_call(
        _deg_reduce_body,
        grid=(1,),
        in_specs=[_full_spec((32, _DROWS, 128))],
        out_specs=_full_spec((_DROWS, 128)),
        out_shape=jax.ShapeDtypeStruct((_DROWS, 128), jnp.float32),
    )(degp)


_sc_mesh = plsc.VectorSubcoreMesh(core_axis_name="c", subcore_axis_name="s")

_IDXROWS = 32 * _NBLK + 16


def _sc_pass_body(vals_hbm, srcb_hbm, dstb_hbm, zeros_hbm, out_hbm,
                  src0, src1, dst0, dst1, row_a, row_b, acc_sh,
                  sem_i0, sem_i1, sem_a, sem_b):
    c = lax.axis_index("c")
    s = lax.axis_index("s")
    base = (c * 16 + s) * _NBLK
    nsb = _NSB

    srcs = (src0, src1)
    dsts = (dst0, dst1)
    isems = (sem_i0, sem_i1)
    rows = (row_a, row_b)
    rsems = (sem_a, sem_b)

    def idx_copies(sb, p):
        rsl = pl.ds(base + sb * _SB, _SB)
        pltpu.async_copy(srcb_hbm.at[rsl], srcs[p], isems[p])
        pltpu.async_copy(dstb_hbm.at[rsl], dsts[p], isems[p])

    def idx_wait(sb, p):
        rsl = pl.ds(base + sb * _SB, _SB)
        pltpu.make_async_copy(srcb_hbm.at[rsl], srcs[p], isems[p]).wait()
        pltpu.make_async_copy(dstb_hbm.at[rsl], dsts[p], isems[p]).wait()

    def gather(p, b):
        pltpu.async_copy(vals_hbm.at[srcs[p].at[b]], rows[b % 2],
                         rsems[b % 2])

    def gather_wait(p, b):
        pltpu.make_async_copy(vals_hbm.at[srcs[p].at[b]], rows[b % 2],
                              rsems[b % 2]).wait()

    def scatter(p, b):
        pltpu.sync_copy(rows[b % 2], acc_sh.at[dsts[p].at[b]], add=True)

    def process_sb(k, p):
        for b in range(_SB):
            if b < _SB - 1:
                gather(p, b + 1)
            else:
                idx_wait(k + 1, 1 - p)
                gather(1 - p, 0)
            gather_wait(p, b)
            if b == _SB - 1:
                idx_copies(k + 2, p)
            scatter(p, b)

    idx_copies(0, 0)
    zrows = pl.ds(s * _RPT, _RPT)
    pltpu.sync_copy(zeros_hbm.at[zrows], acc_sh.at[zrows])
    plsc.subcore_barrier()

    idx_wait(0, 0)
    idx_copies(1, 1)
    gather(0, 0)

    @pl.loop(0, nsb, step=2)
    def _(j):
        process_sb(j, 0)
        process_sb(j + 1, 1)

    gather_wait(0, 0)
    idx_wait(nsb + 1, 1)

    plsc.subcore_barrier()
    pltpu.sync_copy(acc_sh.at[zrows], out_hbm.at[c, zrows])


_sc_pass = pl.kernel(
    _sc_pass_body,
    out_type=jax.ShapeDtypeStruct((2, NPAD, D), jnp.float32),
    mesh=_sc_mesh,
    scratch_types=[
        pltpu.VMEM((_SB, _EB), jnp.int32),
        pltpu.VMEM((_SB, _EB), jnp.int32),
        pltpu.VMEM((_SB, _EB), jnp.int32),
        pltpu.VMEM((_SB, _EB), jnp.int32),
        pltpu.VMEM((_EB, D), jnp.float32),
        pltpu.VMEM((_EB, D), jnp.float32),
        pltpu.VMEM_SHARED((NPAD, D), jnp.float32),
        pltpu.SemaphoreType.DMA,
        pltpu.SemaphoreType.DMA,
        pltpu.SemaphoreType.DMA,
        pltpu.SemaphoreType.DMA,
    ],
)


def _sc_deg_body(dstb_hbm, zeros_hbm, out_hbm, dst_c, deg_tile, sem):
    c = lax.axis_index("c")
    s = lax.axis_index("s")
    wid = c * 16 + s
    base = wid * _NBLK

    pltpu.sync_copy(zeros_hbm.at[pl.ds(0, _DROWS)], deg_tile)
    ones = jnp.ones((16,), jnp.float32)

    @pl.loop(0, _NSB)
    def _(sb):
        pltpu.sync_copy(dstb_hbm.at[pl.ds(base + sb * _SB, _SB)], dst_c)
        for b in range(_SB):
            for k in range(_EB // 16):
                idx = dst_c[b, pl.ds(k * 16, 16)]
                plsc.addupdate_scatter(
                    deg_tile,
                    [lax.shift_right_logical(idx, 7),
                     lax.bitwise_and(idx, 127)],
                    ones)

    pltpu.sync_copy(deg_tile, out_hbm.at[wid])


_sc_cp = pltpu.CompilerParams()
if "needs_layout_passes" in pltpu.CompilerParams.__dataclass_fields__:
    _sc_cp = dataclasses.replace(_sc_cp, needs_layout_passes=False)

_sc_deg = pl.kernel(
    _sc_deg_body,
    out_type=jax.ShapeDtypeStruct((32, _DROWS, 128), jnp.float32),
    mesh=_sc_mesh,
    compiler_params=_sc_cp,
    scratch_types=[
        pltpu.VMEM((_SB, _EB), jnp.int32),
        pltpu.VMEM((_DROWS, 128), jnp.float32),
        pltpu.SemaphoreType.DMA,
    ],
)


def kernel(x, edge_index, W_l1, W_r1, b1, W_l2, W_r2, b2, Wres, bres,
           Wmu, bmu, Wls, bls):
    src = edge_index[0]
    dst = edge_index[1]

    x_pad = jnp.pad(x, ((0, NPAD - N_NODES), (0, 0)))
    b1r = b1.reshape(1, D)
    b2r = b2.reshape(1, D)
    bresr = bres.reshape(1, D)
    wcat = jnp.concatenate([Wmu, Wls], axis=1)
    bcat = jnp.concatenate([bmu, bls]).reshape(1, D)

    pad_e = _EPAD - N_EDGES
    tail = _IDXROWS - 32 * _NBLK
    pad_ids = jnp.arange(pad_e, dtype=jnp.int32)
    pad_src = pad_ids % N_NODES
    pad_dst = pad_ids % (NPAD - N_NODES) + TRASH
    srcb = jnp.concatenate([src, pad_src]).reshape(32 * _NBLK, _EB)
    srcb = jnp.concatenate([srcb, jnp.zeros((tail, _EB), jnp.int32)])
    dstb = jnp.concatenate([dst, pad_dst]).reshape(32 * _NBLK, _EB)
    dstb = jnp.concatenate([dstb, jnp.full((tail, _EB), TRASH, jnp.int32)])
    zeros_big = jnp.zeros((NPAD, D), jnp.float32)

    degp = _sc_deg(dstb, zeros_big)
    deg_col = _deg_reduce(degp).reshape(NPAD, 1)

    agg1p = _sc_pass(x_pad, srcb, dstb, zeros_big)
    h1 = _tc1(agg1p, deg_col, x_pad, W_l1, W_r1, b1r)

    agg2p = _sc_pass(h1, srcb, dstb, zeros_big)
    hn = _tc2(agg2p, deg_col, h1, x_pad, W_l2, W_r2, b2r, Wres, bresr)

    agg3p = _sc_pass(hn, srcb, dstb, zeros_big)
    out = _tc3(agg3p, deg_col, hn, wcat, bcat)

    mu = out[:N_NODES, :64]
    logstd = out[:N_NODES, 64:]
    return (mu, logstd)


# speedup vs baseline: 3.7421x; 1.0091x over previous
import dataclasses
import functools

import jax
import jax.numpy as jnp
from jax import lax
from jax.experimental import pallas as pl
from jax.experimental.pallas import tpu as pltpu
from jax.experimental.pallas import tpu_sc as plsc

N_NODES = 10000
N_EDGES = 320000
NPAD = 10240
TRASH = 10000
D = 128
BLK = 1280
_GRID = NPAD // BLK

_EB = 128
_NBLK = 80
_SB = 8
_NSB = _NBLK // _SB
_EPT = _NBLK * _EB
_EPAD = 32 * _EPT
_RPT = NPAD // 16
_DROWS = NPAD // 128


def _tc1_body(agg_ref, deg_ref, x_ref, wl_ref, wr_ref, b_ref, o_ref):
    agg = agg_ref[0] + agg_ref[1]
    deg = deg_ref[...]
    mean = agg / jnp.maximum(deg, 1.0)
    x = x_ref[...]
    h = jnp.maximum(mean @ wl_ref[...] + x @ wr_ref[...] + b_ref[...], 0.0)
    o_ref[...] = x + h


def _tc2_body(agg_ref, deg_ref, h1_ref, x_ref, wl_ref, wr_ref, b_ref,
              wres_ref, bres_ref, o_ref):
    agg = agg_ref[0] + agg_ref[1]
    deg = deg_ref[...]
    mean = agg / jnp.maximum(deg, 1.0)
    h1 = h1_ref[...]
    h2 = jnp.maximum(mean @ wl_ref[...] + h1 @ wr_ref[...] + b_ref[...], 0.0)
    h2 = h2 + x_ref[...] @ wres_ref[...] + bres_ref[...]
    dinv = lax.rsqrt(deg_ref[...] + 1.0)
    o_ref[...] = h2 * dinv


def _tc3_body(agg_ref, deg_ref, hn_ref, wcat_ref, bcat_ref, mu_ref, ls_ref):
    s = agg_ref[0] + agg_ref[1] + hn_ref[...]
    dinv = lax.rsqrt(deg_ref[...] + 1.0)
    out = (s * dinv) @ wcat_ref[...] + bcat_ref[...]
    mu_ref[...] = out[:, :64]
    ls_ref[...] = out[:, 64:]


def _deg_reduce_body(degp_ref, o_ref):
    o_ref[...] = jnp.sum(degp_ref[...], axis=0)


def _row_spec(width):
    return pl.BlockSpec((BLK, width), lambda i: (i, 0))


def _pair_spec(width):
    return pl.BlockSpec((2, BLK, width), lambda i: (0, i, 0))


def _full_spec(shape):
    return pl.BlockSpec(shape, lambda i: tuple(0 for _ in shape))


def _tc1(aggp, deg, x, wl, wr, b):
    return pl.pallas_call(
        _tc1_body,
        grid=(_GRID,),
        in_specs=[_pair_spec(D), _row_spec(1), _row_spec(D),
                  _full_spec((D, D)), _full_spec((D, D)), _full_spec((1, D))],
        out_specs=_row_spec(D),
        out_shape=jax.ShapeDtypeStruct((NPAD, D), jnp.float32),
    )(aggp, deg, x, wl, wr, b)


def _tc2(aggp, deg, h1, x, wl, wr, b, wres, bres):
    return pl.pallas_call(
        _tc2_body,
        grid=(_GRID,),
        in_specs=[_pair_spec(D), _row_spec(1), _row_spec(D), _row_spec(D),
                  _full_spec((D, D)), _full_spec((D, D)), _full_spec((1, D)),
                  _full_spec((D, D)), _full_spec((1, D))],
        out_specs=_row_spec(D),
        out_shape=jax.ShapeDtypeStruct((NPAD, D), jnp.float32),
    )(aggp, deg, h1, x, wl, wr, b, wres, bres)


def _tc3(aggp, deg, hn, wcat, bcat):
    return pl.pallas_call(
        _tc3_body,
        grid=(_GRID,),
        in_specs=[_pair_spec(D), _row_spec(1), _row_spec(D),
                  _full_spec((D, D)), _full_spec((1, D))],
        out_specs=(_row_spec(64), _row_spec(64)),
        out_shape=(jax.ShapeDtypeStruct((N_NODES, 64), jnp.float32),
                   jax.ShapeDtypeStruct((N_NODES, 64), jnp.float32)),
    )(aggp, deg, hn, wcat, bcat)


def _deg_reduce(degp):
    return pl.pallas_call(
        _deg_reduce_body,
        grid=(1,),
        in_specs=[_full_spec((32, _DROWS, 128))],
        out_specs=_full_spec((_DROWS, 128)),
        out_shape=jax.ShapeDtypeStruct((_DROWS, 128), jnp.float32),
    )(degp)


_sc_mesh = plsc.VectorSubcoreMesh(core_axis_name="c", subcore_axis_name="s")

_IDXROWS = 32 * _NBLK + 16


def _sc_pass_body(vals_hbm, srcb_hbm, dstb_hbm, zeros_hbm, out_hbm,
                  src0, src1, dst0, dst1, row_a, row_b, acc_sh,
                  sem_i0, sem_i1, sem_a, sem_b):
    c = lax.axis_index("c")
    s = lax.axis_index("s")
    base = (c * 16 + s) * _NBLK
    nsb = _NSB

    srcs = (src0, src1)
    dsts = (dst0, dst1)
    isems = (sem_i0, sem_i1)
    rows = (row_a, row_b)
    rsems = (sem_a, sem_b)

    def idx_copies(sb, p):
        rsl = pl.ds(base + sb * _SB, _SB)
        pltpu.async_copy(srcb_hbm.at[rsl], srcs[p], isems[p])
        pltpu.async_copy(dstb_hbm.at[rsl], dsts[p], isems[p])

    def idx_wait(sb, p):
        rsl = pl.ds(base + sb * _SB, _SB)
        pltpu.make_async_copy(srcb_hbm.at[rsl], srcs[p], isems[p]).wait()
        pltpu.make_async_copy(dstb_hbm.at[rsl], dsts[p], isems[p]).wait()

    def gather(p, b):
        pltpu.async_copy(vals_hbm.at[srcs[p].at[b]], rows[b % 2],
                         rsems[b % 2])

    def gather_wait(p, b):
        pltpu.make_async_copy(vals_hbm.at[srcs[p].at[b]], rows[b % 2],
                              rsems[b % 2]).wait()

    def scatter(p, b):
        pltpu.sync_copy(rows[b % 2], acc_sh.at[dsts[p].at[b]], add=True)

    def process_sb(k, p):
        for b in range(_SB):
            if b < _SB - 1:
                gather(p, b + 1)
            else:
                idx_wait(k + 1, 1 - p)
                gather(1 - p, 0)
            gather_wait(p, b)
            if b == _SB - 1:
                idx_copies(k + 2, p)
            scatter(p, b)

    idx_copies(0, 0)
    zrows = pl.ds(s * _RPT, _RPT)
    pltpu.sync_copy(zeros_hbm.at[zrows], acc_sh.at[zrows])
    plsc.subcore_barrier()

    idx_wait(0, 0)
    idx_copies(1, 1)
    gather(0, 0)

    @pl.loop(0, nsb, step=2)
    def _(j):
        process_sb(j, 0)
        process_sb(j + 1, 1)

    gather_wait(0, 0)
    idx_wait(nsb + 1, 1)

    plsc.subcore_barrier()
    pltpu.sync_copy(acc_sh.at[zrows], out_hbm.at[c, zrows])


_sc_pass = pl.kernel(
    _sc_pass_body,
    out_type=jax.ShapeDtypeStruct((2, NPAD, D), jnp.float32),
    mesh=_sc_mesh,
    scratch_types=[
        pltpu.VMEM((_SB, _EB), jnp.int32),
        pltpu.VMEM((_SB, _EB), jnp.int32),
        pltpu.VMEM((_SB, _EB), jnp.int32),
        pltpu.VMEM((_SB, _EB), jnp.int32),
        pltpu.VMEM((_EB, D), jnp.float32),
        pltpu.VMEM((_EB, D), jnp.float32),
        pltpu.VMEM_SHARED((NPAD, D), jnp.float32),
        pltpu.SemaphoreType.DMA,
        pltpu.SemaphoreType.DMA,
        pltpu.SemaphoreType.DMA,
        pltpu.SemaphoreType.DMA,
    ],
)


def _sc_deg_body(dstb_hbm, zeros_hbm, out_hbm, dst_c, deg_tile, sem):
    c = lax.axis_index("c")
    s = lax.axis_index("s")
    wid = c * 16 + s
    base = wid * _NBLK

    pltpu.sync_copy(zeros_hbm.at[pl.ds(0, _DROWS)], deg_tile)
    ones = jnp.ones((16,), jnp.float32)

    @pl.loop(0, _NSB)
    def _(sb):
        pltpu.sync_copy(dstb_hbm.at[pl.ds(base + sb * _SB, _SB)], dst_c)
        for b in range(_SB):
            for k in range(_EB // 16):
                idx = dst_c[b, pl.ds(k * 16, 16)]
                plsc.addupdate_scatter(
                    deg_tile,
                    [lax.shift_right_logical(idx, 7),
                     lax.bitwise_and(idx, 127)],
                    ones)

    pltpu.sync_copy(deg_tile, out_hbm.at[wid])


_sc_cp = pltpu.CompilerParams()
if "needs_layout_passes" in pltpu.CompilerParams.__dataclass_fields__:
    _sc_cp = dataclasses.replace(_sc_cp, needs_layout_passes=False)

_sc_deg = pl.kernel(
    _sc_deg_body,
    out_type=jax.ShapeDtypeStruct((32, _DROWS, 128), jnp.float32),
    mesh=_sc_mesh,
    compiler_params=_sc_cp,
    scratch_types=[
        pltpu.VMEM((_SB, _EB), jnp.int32),
        pltpu.VMEM((_DROWS, 128), jnp.float32),
        pltpu.SemaphoreType.DMA,
    ],
)


def kernel(x, edge_index, W_l1, W_r1, b1, W_l2, W_r2, b2, Wres, bres,
           Wmu, bmu, Wls, bls):
    src = edge_index[0]
    dst = edge_index[1]

    x_pad = jnp.pad(x, ((0, NPAD - N_NODES), (0, 0)))
    b1r = b1.reshape(1, D)
    b2r = b2.reshape(1, D)
    bresr = bres.reshape(1, D)
    wcat = jnp.concatenate([Wmu, Wls], axis=1)
    bcat = jnp.concatenate([bmu, bls]).reshape(1, D)

    pad_e = _EPAD - N_EDGES
    tail = _IDXROWS - 32 * _NBLK
    pad_ids = jnp.arange(pad_e, dtype=jnp.int32)
    pad_src = pad_ids % N_NODES
    pad_dst = pad_ids % (NPAD - N_NODES) + TRASH
    srcb = jnp.concatenate([src, pad_src]).reshape(32 * _NBLK, _EB)
    srcb = jnp.concatenate([srcb, jnp.zeros((tail, _EB), jnp.int32)])
    dstb = jnp.concatenate([dst, pad_dst]).reshape(32 * _NBLK, _EB)
    dstb = jnp.concatenate([dstb, jnp.full((tail, _EB), TRASH, jnp.int32)])
    zeros_big = jnp.zeros((NPAD, D), jnp.float32)

    degp = _sc_deg(dstb, zeros_big)
    deg_col = _deg_reduce(degp).reshape(NPAD, 1)

    agg1p = _sc_pass(x_pad, srcb, dstb, zeros_big)
    h1 = _tc1(agg1p, deg_col, x_pad, W_l1, W_r1, b1r)

    agg2p = _sc_pass(h1, srcb, dstb, zeros_big)
    hn = _tc2(agg2p, deg_col, h1, x_pad, W_l2, W_r2, b2r, Wres, bresr)

    agg3p = _sc_pass(hn, srcb, dstb, zeros_big)
    mu, logstd = _tc3(agg3p, deg_col, hn, wcat, bcat)
    return (mu, logstd)


# speedup vs baseline: 3.7433x; 1.0003x over previous
import dataclasses
import functools

import jax
import jax.numpy as jnp
from jax import lax
from jax.experimental import pallas as pl
from jax.experimental.pallas import tpu as pltpu
from jax.experimental.pallas import tpu_sc as plsc

N_NODES = 10000
N_EDGES = 320000
NPAD = 10240
TRASH = 10000
D = 128
BLK = 1280
_GRID = NPAD // BLK

_EB = 128
_NBLK = 80
_SB = 8
_NSB = _NBLK // _SB
_EPT = _NBLK * _EB
_EPAD = 32 * _EPT
_RPT = NPAD // 16
_DROWS = NPAD // 128


def _tc1_body(agg_ref, deg_ref, x_ref, wl_ref, wr_ref, b_ref, o_ref):
    agg = agg_ref[0] + agg_ref[1]
    deg = deg_ref[...]
    mean = agg / jnp.maximum(deg, 1.0)
    x = x_ref[...]
    h = jnp.maximum(mean @ wl_ref[...] + x @ wr_ref[...] + b_ref[...], 0.0)
    o_ref[...] = x + h


def _tc2_body(agg_ref, deg_ref, h1_ref, x_ref, wl_ref, wr_ref, b_ref,
              wres_ref, bres_ref, o_ref):
    agg = agg_ref[0] + agg_ref[1]
    deg = deg_ref[...]
    mean = agg / jnp.maximum(deg, 1.0)
    h1 = h1_ref[...]
    h2 = jnp.maximum(mean @ wl_ref[...] + h1 @ wr_ref[...] + b_ref[...], 0.0)
    h2 = h2 + x_ref[...] @ wres_ref[...] + bres_ref[...]
    dinv = lax.rsqrt(deg_ref[...] + 1.0)
    o_ref[...] = h2 * dinv


def _tc3_body(agg_ref, deg_ref, hn_ref, wcat_ref, bcat_ref, mu_ref, ls_ref):
    s = agg_ref[0] + agg_ref[1] + hn_ref[...]
    dinv = lax.rsqrt(deg_ref[...] + 1.0)
    out = (s * dinv) @ wcat_ref[...] + bcat_ref[...]
    mu_ref[...] = out[:, :64]
    ls_ref[...] = out[:, 64:]


def _deg_reduce_body(degp_ref, o_ref):
    o_ref[...] = jnp.sum(degp_ref[...], axis=0)


def _row_spec(width):
    return pl.BlockSpec((BLK, width), lambda i: (i, 0))


def _pair_spec(width):
    return pl.BlockSpec((2, BLK, width), lambda i: (0, i, 0))


def _full_spec(shape):
    return pl.BlockSpec(shape, lambda i: tuple(0 for _ in shape))


def _tc1(aggp, deg, x, wl, wr, b):
    return pl.pallas_call(
        _tc1_body,
        grid=(_GRID,),
        in_specs=[_pair_spec(D), _row_spec(1), _row_spec(D),
                  _full_spec((D, D)), _full_spec((D, D)), _full_spec((1, D))],
        out_specs=_row_spec(D),
        out_shape=jax.ShapeDtypeStruct((N_NODES, D), jnp.float32),
    )(aggp, deg, x, wl, wr, b)


def _tc2(aggp, deg, h1, x, wl, wr, b, wres, bres):
    return pl.pallas_call(
        _tc2_body,
        grid=(_GRID,),
        in_specs=[_pair_spec(D), _row_spec(1), _row_spec(D), _row_spec(D),
                  _full_spec((D, D)), _full_spec((D, D)), _full_spec((1, D)),
                  _full_spec((D, D)), _full_spec((1, D))],
        out_specs=_row_spec(D),
        out_shape=jax.ShapeDtypeStruct((N_NODES, D), jnp.float32),
    )(aggp, deg, h1, x, wl, wr, b, wres, bres)


def _tc3(aggp, deg, hn, wcat, bcat):
    return pl.pallas_call(
        _tc3_body,
        grid=(_GRID,),
        in_specs=[_pair_spec(D), _row_spec(1), _row_spec(D),
                  _full_spec((D, D)), _full_spec((1, D))],
        out_specs=(_row_spec(64), _row_spec(64)),
        out_shape=(jax.ShapeDtypeStruct((N_NODES, 64), jnp.float32),
                   jax.ShapeDtypeStruct((N_NODES, 64), jnp.float32)),
    )(aggp, deg, hn, wcat, bcat)


def _deg_reduce(degp):
    return pl.pallas_call(
        _deg_reduce_body,
        grid=(1,),
        in_specs=[_full_spec((32, _DROWS, 128))],
        out_specs=_full_spec((_DROWS, 128)),
        out_shape=jax.ShapeDtypeStruct((_DROWS, 128), jnp.float32),
    )(degp)


_sc_mesh = plsc.VectorSubcoreMesh(core_axis_name="c", subcore_axis_name="s")

_IDXROWS = 32 * _NBLK + 16


def _sc_pass_body(vals_hbm, srcb_hbm, dstb_hbm, zeros_hbm, out_hbm,
                  src0, src1, dst0, dst1, row_a, row_b, acc_sh,
                  sem_i0, sem_i1, sem_a, sem_b):
    c = lax.axis_index("c")
    s = lax.axis_index("s")
    base = (c * 16 + s) * _NBLK
    nsb = _NSB

    srcs = (src0, src1)
    dsts = (dst0, dst1)
    isems = (sem_i0, sem_i1)
    rows = (row_a, row_b)
    rsems = (sem_a, sem_b)

    def idx_copies(sb, p):
        rsl = pl.ds(base + sb * _SB, _SB)
        pltpu.async_copy(srcb_hbm.at[rsl], srcs[p], isems[p])
        pltpu.async_copy(dstb_hbm.at[rsl], dsts[p], isems[p])

    def idx_wait(sb, p):
        rsl = pl.ds(base + sb * _SB, _SB)
        pltpu.make_async_copy(srcb_hbm.at[rsl], srcs[p], isems[p]).wait()
        pltpu.make_async_copy(dstb_hbm.at[rsl], dsts[p], isems[p]).wait()

    def gather(p, b):
        pltpu.async_copy(vals_hbm.at[srcs[p].at[b]], rows[b % 2],
                         rsems[b % 2])

    def gather_wait(p, b):
        pltpu.make_async_copy(vals_hbm.at[srcs[p].at[b]], rows[b % 2],
                              rsems[b % 2]).wait()

    def scatter(p, b):
        pltpu.sync_copy(rows[b % 2], acc_sh.at[dsts[p].at[b]], add=True)

    def process_sb(k, p):
        for b in range(_SB):
            if b < _SB - 1:
                gather(p, b + 1)
            else:
                idx_wait(k + 1, 1 - p)
                gather(1 - p, 0)
            gather_wait(p, b)
            if b == _SB - 1:
                idx_copies(k + 2, p)
            scatter(p, b)

    idx_copies(0, 0)
    zrows = pl.ds(s * _RPT, _RPT)
    pltpu.sync_copy(zeros_hbm.at[zrows], acc_sh.at[zrows])
    plsc.subcore_barrier()

    idx_wait(0, 0)
    idx_copies(1, 1)
    gather(0, 0)

    @pl.loop(0, nsb, step=2)
    def _(j):
        process_sb(j, 0)
        process_sb(j + 1, 1)

    gather_wait(0, 0)
    idx_wait(nsb + 1, 1)

    plsc.subcore_barrier()
    pltpu.sync_copy(acc_sh.at[zrows], out_hbm.at[c, zrows])


_sc_pass = pl.kernel(
    _sc_pass_body,
    out_type=jax.ShapeDtypeStruct((2, NPAD, D), jnp.float32),
    mesh=_sc_mesh,
    scratch_types=[
        pltpu.VMEM((_SB, _EB), jnp.int32),
        pltpu.VMEM((_SB, _EB), jnp.int32),
        pltpu.VMEM((_SB, _EB), jnp.int32),
        pltpu.VMEM((_SB, _EB), jnp.int32),
        pltpu.VMEM((_EB, D), jnp.float32),
        pltpu.VMEM((_EB, D), jnp.float32),
        pltpu.VMEM_SHARED((NPAD, D), jnp.float32),
        pltpu.SemaphoreType.DMA,
        pltpu.SemaphoreType.DMA,
        pltpu.SemaphoreType.DMA,
        pltpu.SemaphoreType.DMA,
    ],
)


def _sc_deg_body(dstb_hbm, zeros_hbm, out_hbm, dst_c, deg_tile, sem):
    c = lax.axis_index("c")
    s = lax.axis_index("s")
    wid = c * 16 + s
    base = wid * _NBLK

    pltpu.sync_copy(zeros_hbm.at[pl.ds(0, _DROWS)], deg_tile)
    ones = jnp.ones((16,), jnp.float32)

    @pl.loop(0, _NSB)
    def _(sb):
        pltpu.sync_copy(dstb_hbm.at[pl.ds(base + sb * _SB, _SB)], dst_c)
        for b in range(_SB):
            for k in range(_EB // 16):
                idx = dst_c[b, pl.ds(k * 16, 16)]
                plsc.addupdate_scatter(
                    deg_tile,
                    [lax.shift_right_logical(idx, 7),
                     lax.bitwise_and(idx, 127)],
                    ones)

    pltpu.sync_copy(deg_tile, out_hbm.at[wid])


_sc_cp = pltpu.CompilerParams()
if "needs_layout_passes" in pltpu.CompilerParams.__dataclass_fields__:
    _sc_cp = dataclasses.replace(_sc_cp, needs_layout_passes=False)

_sc_deg = pl.kernel(
    _sc_deg_body,
    out_type=jax.ShapeDtypeStruct((32, _DROWS, 128), jnp.float32),
    mesh=_sc_mesh,
    compiler_params=_sc_cp,
    scratch_types=[
        pltpu.VMEM((_SB, _EB), jnp.int32),
        pltpu.VMEM((_DROWS, 128), jnp.float32),
        pltpu.SemaphoreType.DMA,
    ],
)


def kernel(x, edge_index, W_l1, W_r1, b1, W_l2, W_r2, b2, Wres, bres,
           Wmu, bmu, Wls, bls):
    src = edge_index[0]
    dst = edge_index[1]

    b1r = b1.reshape(1, D)
    b2r = b2.reshape(1, D)
    bresr = bres.reshape(1, D)
    wcat = jnp.concatenate([Wmu, Wls], axis=1)
    bcat = jnp.concatenate([bmu, bls]).reshape(1, D)

    pad_e = _EPAD - N_EDGES
    tail = _IDXROWS - 32 * _NBLK
    pad_ids = jnp.arange(pad_e, dtype=jnp.int32)
    pad_src = pad_ids % N_NODES
    pad_dst = pad_ids % (NPAD - N_NODES) + TRASH
    srcb = jnp.concatenate([src, pad_src]).reshape(32 * _NBLK, _EB)
    srcb = jnp.concatenate([srcb, jnp.zeros((tail, _EB), jnp.int32)])
    dstb = jnp.concatenate([dst, pad_dst]).reshape(32 * _NBLK, _EB)
    dstb = jnp.concatenate([dstb, jnp.full((tail, _EB), TRASH, jnp.int32)])
    zeros_big = jnp.zeros((NPAD, D), jnp.float32)

    degp = _sc_deg(dstb, zeros_big)
    deg_col = _deg_reduce(degp).reshape(NPAD, 1)

    agg1p = _sc_pass(x, srcb, dstb, zeros_big)
    h1 = _tc1(agg1p, deg_col, x, W_l1, W_r1, b1r)

    agg2p = _sc_pass(h1, srcb, dstb, zeros_big)
    hn = _tc2(agg2p, deg_col, h1, x, W_l2, W_r2, b2r, Wres, bresr)

    agg3p = _sc_pass(hn, srcb, dstb, zeros_big)
    mu, logstd = _tc3(agg3p, deg_col, hn, wcat, bcat)
    return (mu, logstd)


# speedup vs baseline: 3.7514x; 1.0022x over previous
import dataclasses
import functools

import jax
import jax.numpy as jnp
from jax import lax
from jax.experimental import pallas as pl
from jax.experimental.pallas import tpu as pltpu
from jax.experimental.pallas import tpu_sc as plsc

N_NODES = 10000
N_EDGES = 320000
NPAD = 10240
TRASH = 10000
D = 128
BLK = 1280
_GRID = NPAD // BLK

_EB = 128
_NBLK = 80
_SB = 8
_NSB = _NBLK // _SB
_EPT = _NBLK * _EB
_EPAD = 32 * _EPT
_RPT = NPAD // 16
_DROWS = NPAD // 128


def _mm(a, w_ref):
    return lax.dot(a, w_ref[...], precision=lax.Precision.DEFAULT)


def _tc1_body(agg_ref, deg_ref, x_ref, wl_ref, wr_ref, b_ref, o_ref):
    agg = agg_ref[0] + agg_ref[1]
    deg = deg_ref[...]
    mean = agg / jnp.maximum(deg, 1.0)
    x = x_ref[...]
    h = jnp.maximum(_mm(mean, wl_ref) + _mm(x, wr_ref) + b_ref[...], 0.0)
    o_ref[...] = x + h


def _tc2_body(agg_ref, deg_ref, h1_ref, x_ref, wl_ref, wr_ref, b_ref,
              wres_ref, bres_ref, o_ref):
    agg = agg_ref[0] + agg_ref[1]
    deg = deg_ref[...]
    mean = agg / jnp.maximum(deg, 1.0)
    h1 = h1_ref[...]
    h2 = jnp.maximum(_mm(mean, wl_ref) + _mm(h1, wr_ref) + b_ref[...], 0.0)
    h2 = h2 + _mm(x_ref[...], wres_ref) + bres_ref[...]
    dinv = lax.rsqrt(deg_ref[...] + 1.0)
    o_ref[...] = h2 * dinv


def _tc3_body(agg_ref, deg_ref, hn_ref, wcat_ref, bcat_ref, mu_ref, ls_ref):
    s = agg_ref[0] + agg_ref[1] + hn_ref[...]
    dinv = lax.rsqrt(deg_ref[...] + 1.0)
    out = _mm(s * dinv, wcat_ref) + bcat_ref[...]
    mu_ref[...] = out[:, :64]
    ls_ref[...] = out[:, 64:]


def _deg_reduce_body(degp_ref, o_ref):
    o_ref[...] = jnp.sum(degp_ref[...], axis=0)


def _row_spec(width):
    return pl.BlockSpec((BLK, width), lambda i: (i, 0))


def _pair_spec(width):
    return pl.BlockSpec((2, BLK, width), lambda i: (0, i, 0))


def _full_spec(shape):
    return pl.BlockSpec(shape, lambda i: tuple(0 for _ in shape))


def _tc1(aggp, deg, x, wl, wr, b):
    return pl.pallas_call(
        _tc1_body,
        grid=(_GRID,),
        in_specs=[_pair_spec(D), _row_spec(1), _row_spec(D),
                  _full_spec((D, D)), _full_spec((D, D)), _full_spec((1, D))],
        out_specs=_row_spec(D),
        out_shape=jax.ShapeDtypeStruct((N_NODES, D), jnp.float32),
    )(aggp, deg, x, wl, wr, b)


def _tc2(aggp, deg, h1, x, wl, wr, b, wres, bres):
    return pl.pallas_call(
        _tc2_body,
        grid=(_GRID,),
        in_specs=[_pair_spec(D), _row_spec(1), _row_spec(D), _row_spec(D),
                  _full_spec((D, D)), _full_spec((D, D)), _full_spec((1, D)),
                  _full_spec((D, D)), _full_spec((1, D))],
        out_specs=_row_spec(D),
        out_shape=jax.ShapeDtypeStruct((N_NODES, D), jnp.float32),
    )(aggp, deg, h1, x, wl, wr, b, wres, bres)


def _tc3(aggp, deg, hn, wcat, bcat):
    return pl.pallas_call(
        _tc3_body,
        grid=(_GRID,),
        in_specs=[_pair_spec(D), _row_spec(1), _row_spec(D),
                  _full_spec((D, D)), _full_spec((1, D))],
        out_specs=(_row_spec(64), _row_spec(64)),
        out_shape=(jax.ShapeDtypeStruct((N_NODES, 64), jnp.float32),
                   jax.ShapeDtypeStruct((N_NODES, 64), jnp.float32)),
    )(aggp, deg, hn, wcat, bcat)


def _deg_reduce(degp):
    return pl.pallas_call(
        _deg_reduce_body,
        grid=(1,),
        in_specs=[_full_spec((32, _DROWS, 128))],
        out_specs=_full_spec((_DROWS, 128)),
        out_shape=jax.ShapeDtypeStruct((_DROWS, 128), jnp.float32),
    )(degp)


_sc_mesh = plsc.VectorSubcoreMesh(core_axis_name="c", subcore_axis_name="s")

_IDXROWS = 32 * _NBLK + 16


def _sc_pass_body(vals_hbm, srcb_hbm, dstb_hbm, zeros_hbm, out_hbm,
                  src0, src1, dst0, dst1, row_a, row_b, acc_sh,
                  sem_i0, sem_i1, sem_a, sem_b):
    c = lax.axis_index("c")
    s = lax.axis_index("s")
    base = (c * 16 + s) * _NBLK
    nsb = _NSB

    srcs = (src0, src1)
    dsts = (dst0, dst1)
    isems = (sem_i0, sem_i1)
    rows = (row_a, row_b)
    rsems = (sem_a, sem_b)

    def idx_copies(sb, p):
        rsl = pl.ds(base + sb * _SB, _SB)
        pltpu.async_copy(srcb_hbm.at[rsl], srcs[p], isems[p])
        pltpu.async_copy(dstb_hbm.at[rsl], dsts[p], isems[p])

    def idx_wait(sb, p):
        rsl = pl.ds(base + sb * _SB, _SB)
        pltpu.make_async_copy(srcb_hbm.at[rsl], srcs[p], isems[p]).wait()
        pltpu.make_async_copy(dstb_hbm.at[rsl], dsts[p], isems[p]).wait()

    def gather(p, b):
        pltpu.async_copy(vals_hbm.at[srcs[p].at[b]], rows[b % 2],
                         rsems[b % 2])

    def gather_wait(p, b):
        pltpu.make_async_copy(vals_hbm.at[srcs[p].at[b]], rows[b % 2],
                              rsems[b % 2]).wait()

    def scatter(p, b):
        pltpu.sync_copy(rows[b % 2], acc_sh.at[dsts[p].at[b]], add=True)

    def process_sb(k, p):
        for b in range(_SB):
            if b < _SB - 1:
                gather(p, b + 1)
            else:
                idx_wait(k + 1, 1 - p)
                gather(1 - p, 0)
            gather_wait(p, b)
            if b == _SB - 1:
                idx_copies(k + 2, p)
            scatter(p, b)

    idx_copies(0, 0)
    zrows = pl.ds(s * _RPT, _RPT)
    pltpu.sync_copy(zeros_hbm.at[zrows], acc_sh.at[zrows])
    plsc.subcore_barrier()

    idx_wait(0, 0)
    idx_copies(1, 1)
    gather(0, 0)

    @pl.loop(0, nsb, step=2)
    def _(j):
        process_sb(j, 0)
        process_sb(j + 1, 1)

    gather_wait(0, 0)
    idx_wait(nsb + 1, 1)

    plsc.subcore_barrier()
    pltpu.sync_copy(acc_sh.at[zrows], out_hbm.at[c, zrows])


_sc_pass = pl.kernel(
    _sc_pass_body,
    out_type=jax.ShapeDtypeStruct((2, NPAD, D), jnp.float32),
    mesh=_sc_mesh,
    scratch_types=[
        pltpu.VMEM((_SB, _EB), jnp.int32),
        pltpu.VMEM((_SB, _EB), jnp.int32),
        pltpu.VMEM((_SB, _EB), jnp.int32),
        pltpu.VMEM((_SB, _EB), jnp.int32),
        pltpu.VMEM((_EB, D), jnp.float32),
        pltpu.VMEM((_EB, D), jnp.float32),
        pltpu.VMEM_SHARED((NPAD, D), jnp.float32),
        pltpu.SemaphoreType.DMA,
        pltpu.SemaphoreType.DMA,
        pltpu.SemaphoreType.DMA,
        pltpu.SemaphoreType.DMA,
    ],
)


def _sc_deg_body(dstb_hbm, zeros_hbm, out_hbm, dst_c, deg_tile, sem):
    c = lax.axis_index("c")
    s = lax.axis_index("s")
    wid = c * 16 + s
    base = wid * _NBLK

    pltpu.sync_copy(zeros_hbm.at[pl.ds(0, _DROWS)], deg_tile)
    ones = jnp.ones((16,), jnp.float32)

    @pl.loop(0, _NSB)
    def _(sb):
        pltpu.sync_copy(dstb_hbm.at[pl.ds(base + sb * _SB, _SB)], dst_c)
        for b in range(_SB):
            for k in range(_EB // 16):
                idx = dst_c[b, pl.ds(k * 16, 16)]
                plsc.addupdate_scatter(
                    deg_tile,
                    [lax.shift_right_logical(idx, 7),
                     lax.bitwise_and(idx, 127)],
                    ones)

    pltpu.sync_copy(deg_tile, out_hbm.at[wid])


_sc_cp = pltpu.CompilerParams()
if "needs_layout_passes" in pltpu.CompilerParams.__dataclass_fields__:
    _sc_cp = dataclasses.replace(_sc_cp, needs_layout_passes=False)

_sc_deg = pl.kernel(
    _sc_deg_body,
    out_type=jax.ShapeDtypeStruct((32, _DROWS, 128), jnp.float32),
    mesh=_sc_mesh,
    compiler_params=_sc_cp,
    scratch_types=[
        pltpu.VMEM((_SB, _EB), jnp.int32),
        pltpu.VMEM((_DROWS, 128), jnp.float32),
        pltpu.SemaphoreType.DMA,
    ],
)


def kernel(x, edge_index, W_l1, W_r1, b1, W_l2, W_r2, b2, Wres, bres,
           Wmu, bmu, Wls, bls):
    src = edge_index[0]
    dst = edge_index[1]

    b1r = b1.reshape(1, D)
    b2r = b2.reshape(1, D)
    bresr = bres.reshape(1, D)
    wcat = jnp.concatenate([Wmu, Wls], axis=1)
    bcat = jnp.concatenate([bmu, bls]).reshape(1, D)

    pad_e = _EPAD - N_EDGES
    tail = _IDXROWS - 32 * _NBLK
    pad_ids = jnp.arange(pad_e, dtype=jnp.int32)
    pad_src = pad_ids % N_NODES
    pad_dst = pad_ids % (NPAD - N_NODES) + TRASH
    srcb = jnp.concatenate([src, pad_src]).reshape(32 * _NBLK, _EB)
    srcb = jnp.concatenate([srcb, jnp.zeros((tail, _EB), jnp.int32)])
    dstb = jnp.concatenate([dst, pad_dst]).reshape(32 * _NBLK, _EB)
    dstb = jnp.concatenate([dstb, jnp.full((tail, _EB), TRASH, jnp.int32)])
    zeros_big = jnp.zeros((NPAD, D), jnp.float32)

    degp = _sc_deg(dstb, zeros_big)
    deg_col = _deg_reduce(degp).reshape(NPAD, 1)

    agg1p = _sc_pass(x, srcb, dstb, zeros_big)
    h1 = _tc1(agg1p, deg_col, x, W_l1, W_r1, b1r)

    agg2p = _sc_pass(h1, srcb, dstb, zeros_big)
    hn = _tc2(agg2p, deg_col, h1, x, W_l2, W_r2, b2r, Wres, bresr)

    agg3p = _sc_pass(hn, srcb, dstb, zeros_big)
    mu, logstd = _tc3(agg3p, deg_col, hn, wcat, bcat)
    return (mu, logstd)


# speedup vs baseline: 3.7531x; 1.0005x over previous
import dataclasses
import functools

import jax
import jax.numpy as jnp
from jax import lax
from jax.experimental import pallas as pl
from jax.experimental.pallas import tpu as pltpu
from jax.experimental.pallas import tpu_sc as plsc

N_NODES = 10000
N_EDGES = 320000
NPAD = 10240
TRASH = 10000
D = 128
BLK = 1280
_GRID = NPAD // BLK

_EB = 128
_NBLK = 80
_SB = 8
_NSB = _NBLK // _SB
_EPT = _NBLK * _EB
_EPAD = 32 * _EPT
_RPT = NPAD // 16
_DROWS = NPAD // 128


def _mm(a, w_ref):
    return lax.dot(a, w_ref[...], precision=lax.Precision.DEFAULT)


def _tc1_body(agg_ref, deg_ref, x_ref, wl_ref, wr_ref, b_ref, o_ref):
    agg = agg_ref[0] + agg_ref[1]
    deg = deg_ref[...]
    mean = agg / jnp.maximum(deg, 1.0)
    x = x_ref[...]
    h = jnp.maximum(_mm(mean, wl_ref) + _mm(x, wr_ref) + b_ref[...], 0.0)
    o_ref[...] = x + h


def _tc2_body(agg_ref, deg_ref, h1_ref, x_ref, wl_ref, wr_ref, b_ref,
              wres_ref, bres_ref, o_ref):
    agg = agg_ref[0] + agg_ref[1]
    deg = deg_ref[...]
    mean = agg / jnp.maximum(deg, 1.0)
    h1 = h1_ref[...]
    h2 = jnp.maximum(_mm(mean, wl_ref) + _mm(h1, wr_ref) + b_ref[...], 0.0)
    h2 = h2 + _mm(x_ref[...], wres_ref) + bres_ref[...]
    dinv = lax.rsqrt(deg_ref[...] + 1.0)
    o_ref[...] = h2 * dinv


def _tc3_body(agg_ref, deg_ref, hn_ref, wcat_ref, bcat_ref, mu_ref, ls_ref):
    s = agg_ref[0] + agg_ref[1] + hn_ref[...]
    dinv = lax.rsqrt(deg_ref[...] + 1.0)
    out = _mm(s * dinv, wcat_ref) + bcat_ref[...]
    mu_ref[...] = out[:, :64]
    ls_ref[...] = out[:, 64:]


def _deg_reduce_body(degp_ref, o_ref):
    o_ref[...] = jnp.sum(degp_ref[...], axis=0)


def _row_spec(width):
    return pl.BlockSpec((BLK, width), lambda i: (i, 0))


def _pair_spec(width):
    return pl.BlockSpec((2, BLK, width), lambda i: (0, i, 0))


def _full_spec(shape):
    return pl.BlockSpec(shape, lambda i: tuple(0 for _ in shape))


def _tc1(aggp, deg, x, wl, wr, b):
    return pl.pallas_call(
        _tc1_body,
        grid=(_GRID,),
        in_specs=[_pair_spec(D), _row_spec(1), _row_spec(D),
                  _full_spec((D, D)), _full_spec((D, D)), _full_spec((1, D))],
        out_specs=_row_spec(D),
        out_shape=jax.ShapeDtypeStruct((N_NODES, D), jnp.float32),
    )(aggp, deg, x, wl, wr, b)


def _tc2(aggp, deg, h1, x, wl, wr, b, wres, bres):
    return pl.pallas_call(
        _tc2_body,
        grid=(_GRID,),
        in_specs=[_pair_spec(D), _row_spec(1), _row_spec(D), _row_spec(D),
                  _full_spec((D, D)), _full_spec((D, D)), _full_spec((1, D)),
                  _full_spec((D, D)), _full_spec((1, D))],
        out_specs=_row_spec(D),
        out_shape=jax.ShapeDtypeStruct((N_NODES, D), jnp.float32),
    )(aggp, deg, h1, x, wl, wr, b, wres, bres)


def _tc3(aggp, deg, hn, wcat, bcat):
    return pl.pallas_call(
        _tc3_body,
        grid=(_GRID,),
        in_specs=[_pair_spec(D), _row_spec(1), _row_spec(D),
                  _full_spec((D, D)), _full_spec((1, D))],
        out_specs=(_row_spec(64), _row_spec(64)),
        out_shape=(jax.ShapeDtypeStruct((N_NODES, 64), jnp.float32),
                   jax.ShapeDtypeStruct((N_NODES, 64), jnp.float32)),
    )(aggp, deg, hn, wcat, bcat)


def _deg_reduce(degp):
    return pl.pallas_call(
        _deg_reduce_body,
        grid=(1,),
        in_specs=[_full_spec((32, _DROWS, 128))],
        out_specs=_full_spec((_DROWS, 128)),
        out_shape=jax.ShapeDtypeStruct((_DROWS, 128), jnp.float32),
    )(degp)


_sc_mesh = plsc.VectorSubcoreMesh(core_axis_name="c", subcore_axis_name="s")

_IDXROWS = 32 * _NBLK + 2 * _SB


def _sc_pass_body(vals_hbm, srcb_hbm, dstb_hbm, zeros_hbm, out_hbm,
                  src0, src1, dst0, dst1, row_a, row_b, acc_sh,
                  sem_i0, sem_i1, sem_a, sem_b):
    c = lax.axis_index("c")
    s = lax.axis_index("s")
    base = (c * 16 + s) * _NBLK
    nsb = _NSB

    srcs = (src0, src1)
    dsts = (dst0, dst1)
    isems = (sem_i0, sem_i1)
    rows = (row_a, row_b)
    rsems = (sem_a, sem_b)

    def idx_copies(sb, p):
        rsl = pl.ds(base + sb * _SB, _SB)
        pltpu.async_copy(srcb_hbm.at[rsl], srcs[p], isems[p])
        pltpu.async_copy(dstb_hbm.at[rsl], dsts[p], isems[p])

    def idx_wait(sb, p):
        rsl = pl.ds(base + sb * _SB, _SB)
        pltpu.make_async_copy(srcb_hbm.at[rsl], srcs[p], isems[p]).wait()
        pltpu.make_async_copy(dstb_hbm.at[rsl], dsts[p], isems[p]).wait()

    def gather(p, b):
        pltpu.async_copy(vals_hbm.at[srcs[p].at[b]], rows[b % 2],
                         rsems[b % 2])

    def gather_wait(p, b):
        pltpu.make_async_copy(vals_hbm.at[srcs[p].at[b]], rows[b % 2],
                              rsems[b % 2]).wait()

    def scatter(p, b):
        pltpu.sync_copy(rows[b % 2], acc_sh.at[dsts[p].at[b]], add=True)

    def process_sb(k, p):
        for b in range(_SB):
            if b < _SB - 1:
                gather(p, b + 1)
            else:
                idx_wait(k + 1, 1 - p)
                gather(1 - p, 0)
            gather_wait(p, b)
            if b == _SB - 1:
                idx_copies(k + 2, p)
            scatter(p, b)

    idx_copies(0, 0)
    zrows = pl.ds(s * _RPT, _RPT)
    pltpu.sync_copy(zeros_hbm.at[zrows], acc_sh.at[zrows])
    plsc.subcore_barrier()

    idx_wait(0, 0)
    idx_copies(1, 1)
    gather(0, 0)

    @pl.loop(0, nsb, step=2)
    def _(j):
        process_sb(j, 0)
        process_sb(j + 1, 1)

    gather_wait(0, 0)
    idx_wait(nsb + 1, 1)

    plsc.subcore_barrier()
    pltpu.sync_copy(acc_sh.at[zrows], out_hbm.at[c, zrows])


_sc_pass = pl.kernel(
    _sc_pass_body,
    out_type=jax.ShapeDtypeStruct((2, NPAD, D), jnp.float32),
    mesh=_sc_mesh,
    scratch_types=[
        pltpu.VMEM((_SB, _EB), jnp.int32),
        pltpu.VMEM((_SB, _EB), jnp.int32),
        pltpu.VMEM((_SB, _EB), jnp.int32),
        pltpu.VMEM((_SB, _EB), jnp.int32),
        pltpu.VMEM((_EB, D), jnp.float32),
        pltpu.VMEM((_EB, D), jnp.float32),
        pltpu.VMEM_SHARED((NPAD, D), jnp.float32),
        pltpu.SemaphoreType.DMA,
        pltpu.SemaphoreType.DMA,
        pltpu.SemaphoreType.DMA,
        pltpu.SemaphoreType.DMA,
    ],
)


def _sc_deg_body(dstb_hbm, zeros_hbm, out_hbm, dst_c, deg_tile, sem):
    c = lax.axis_index("c")
    s = lax.axis_index("s")
    wid = c * 16 + s
    base = wid * _NBLK

    pltpu.sync_copy(zeros_hbm.at[pl.ds(0, _DROWS)], deg_tile)
    ones = jnp.ones((16,), jnp.float32)

    @pl.loop(0, _NSB)
    def _(sb):
        pltpu.sync_copy(dstb_hbm.at[pl.ds(base + sb * _SB, _SB)], dst_c)
        for b in range(_SB):
            for k in range(_EB // 16):
                idx = dst_c[b, pl.ds(k * 16, 16)]
                plsc.addupdate_scatter(
                    deg_tile,
                    [lax.shift_right_logical(idx, 7),
                     lax.bitwise_and(idx, 127)],
                    ones)

    pltpu.sync_copy(deg_tile, out_hbm.at[wid])


_sc_cp = pltpu.CompilerParams()
if "needs_layout_passes" in pltpu.CompilerParams.__dataclass_fields__:
    _sc_cp = dataclasses.replace(_sc_cp, needs_layout_passes=False)

_sc_deg = pl.kernel(
    _sc_deg_body,
    out_type=jax.ShapeDtypeStruct((32, _DROWS, 128), jnp.float32),
    mesh=_sc_mesh,
    compiler_params=_sc_cp,
    scratch_types=[
        pltpu.VMEM((_SB, _EB), jnp.int32),
        pltpu.VMEM((_DROWS, 128), jnp.float32),
        pltpu.SemaphoreType.DMA,
    ],
)


def kernel(x, edge_index, W_l1, W_r1, b1, W_l2, W_r2, b2, Wres, bres,
           Wmu, bmu, Wls, bls):
    src = edge_index[0]
    dst = edge_index[1]

    b1r = b1.reshape(1, D)
    b2r = b2.reshape(1, D)
    bresr = bres.reshape(1, D)
    wcat = jnp.concatenate([Wmu, Wls], axis=1)
    bcat = jnp.concatenate([bmu, bls]).reshape(1, D)

    pad_e = _EPAD - N_EDGES
    tail = _IDXROWS - 32 * _NBLK
    pad_ids = jnp.arange(pad_e, dtype=jnp.int32)
    pad_src = pad_ids % N_NODES
    pad_dst = pad_ids % (NPAD - N_NODES) + TRASH
    srcb = jnp.concatenate([src, pad_src]).reshape(32 * _NBLK, _EB)
    srcb = jnp.concatenate([srcb, jnp.zeros((tail, _EB), jnp.int32)])
    dstb = jnp.concatenate([dst, pad_dst]).reshape(32 * _NBLK, _EB)
    dstb = jnp.concatenate([dstb, jnp.full((tail, _EB), TRASH, jnp.int32)])
    zeros_big = jnp.zeros((NPAD, D), jnp.float32)

    degp = _sc_deg(dstb, zeros_big)
    deg_col = _deg_reduce(degp).reshape(NPAD, 1)

    agg1p = _sc_pass(x, srcb, dstb, zeros_big)
    h1 = _tc1(agg1p, deg_col, x, W_l1, W_r1, b1r)

    agg2p = _sc_pass(h1, srcb, dstb, zeros_big)
    hn = _tc2(agg2p, deg_col, h1, x, W_l2, W_r2, b2r, Wres, bresr)

    agg3p = _sc_pass(hn, srcb, dstb, zeros_big)
    mu, logstd = _tc3(agg3p, deg_col, hn, wcat, bcat)
    return (mu, logstd)
